# baseline (speedup 1.0000x reference)
import functools

import jax
import jax.numpy as jnp
from jax import lax
from jax.experimental import pallas as pl
from jax.experimental.pallas import tpu as pltpu
from jax.experimental.pallas import tpu_sc as plsc

N_NODE = 10000
N_RAD = 6
N_L = 10
N_C = 16
N_OUT = 4

SC_LANES = 9984
CHN = 128
NUM_WORKERS = 32
JGROUPS = CHN // 16
JSHIFT = JGROUPS.bit_length() - 1
TC_BN = 512


def _poly(a):
    s1 = a[1] * a[1]
    s2 = a[2] * a[2]
    s3 = a[3] * a[3]
    s4 = a[4] * a[4]
    s5 = a[5] * a[5]
    s6 = a[6] * a[6]
    s7 = a[7] * a[7]
    s8 = a[8] * a[8]
    s9 = a[9] * a[9]
    out1 = s1 + s2 + s3
    out2 = (s4 + s7 + s9) + 2.0 * (s5 + s6 + s8)
    cubes = a[4] * s4 + a[7] * s7 + a[9] * s9
    mixed = a[4] * (s5 + s6) + a[7] * (s5 + s8) + a[9] * (s6 + s8)
    triple = a[5] * a[6] * a[8]
    out3 = cubes + 3.0 * mixed + 6.0 * triple
    return a[0], out1, out2, out3


def _tc_body(x_ref, o_ref):
    a = [x_ref[:, l] for l in range(N_L)]
    o0, o1, o2, o3 = _poly(a)
    o_ref[:, 0] = o0
    o_ref[:, 1] = o1
    o_ref[:, 2] = o2
    o_ref[:, 3] = o3


def _make_tc_call(bn, start, count):
    grid = (count + bn - 1) // bn
    sb = start // bn
    return pl.pallas_call(
        _tc_body,
        grid=(grid,),
        in_specs=[
            pl.BlockSpec((N_RAD, N_L, N_C, bn), lambda i: (0, 0, 0, i + sb)),
        ],
        out_specs=pl.BlockSpec((N_RAD, N_OUT, N_C, bn), lambda i: (0, 0, 0, i)),
        out_shape=jax.ShapeDtypeStruct((N_RAD, N_OUT, N_C, count), jnp.float32),
    )


def _sc_body(x_hbm, o_hbm, xv, ov):
    wid = lax.axis_index("s") * 2 + lax.axis_index("c")
    nchunks = SC_LANES // CHN
    units = N_RAD * nchunks
    my_units = (units - wid + NUM_WORKERS - 1) // NUM_WORKERS

    def unit_body(u, carry):
        unit = wid + u * NUM_WORKERS
        r = unit // nchunks
        n0 = (unit % nchunks) * CHN
        pltpu.sync_copy(x_hbm.at[r, :, :, pl.ds(n0, CHN)], xv)

        @plsc.parallel_loop(0, N_C * JGROUPS, unroll=4)
        def group_body(m):
            c = m >> JSHIFT
            j = (m & (JGROUPS - 1)) * 16
            a = [xv[l, c, pl.ds(j, 16)] for l in range(N_L)]
            o0, o1, o2, o3 = _poly(a)
            ov[0, c, pl.ds(j, 16)] = o0
            ov[1, c, pl.ds(j, 16)] = o1
            ov[2, c, pl.ds(j, 16)] = o2
            ov[3, c, pl.ds(j, 16)] = o3

        pltpu.sync_copy(ov, o_hbm.at[r, :, :, pl.ds(n0, CHN)])
        return carry

    lax.fori_loop(0, my_units, unit_body, 0)


if SC_LANES:
    _sc_call = functools.partial(
        pl.kernel,
        out_type=jax.ShapeDtypeStruct((N_RAD, N_OUT, N_C, SC_LANES), jnp.float32),
        mesh=plsc.VectorSubcoreMesh(core_axis_name="c", subcore_axis_name="s"),
        scratch_types=[
            pltpu.VMEM((N_L, N_C, CHN), jnp.float32),
            pltpu.VMEM((N_OUT, N_C, CHN), jnp.float32),
        ],
    )(_sc_body)

if SC_LANES < N_NODE:
    _tc_bn = TC_BN if SC_LANES % TC_BN == 0 else 128
    _tc_call = _make_tc_call(_tc_bn, SC_LANES, N_NODE - SC_LANES)


@jax.jit
def kernel(node_attr):
    xt = jnp.transpose(node_attr, (1, 2, 3, 0))
    parts = []
    if SC_LANES:
        parts.append(_sc_call(xt))
    if SC_LANES < N_NODE:
        parts.append(_tc_call(xt))
    yt = parts[0] if len(parts) == 1 else jnp.concatenate(parts, axis=3)
    return jnp.transpose(yt, (3, 0, 1, 2))


# speedup vs baseline: 1.7113x; 1.7113x over previous
import functools

import jax
import jax.numpy as jnp
from jax import lax
from jax.experimental import pallas as pl
from jax.experimental.pallas import tpu as pltpu
from jax.experimental.pallas import tpu_sc as plsc

N_NODE = 10000
N_RAD = 6
N_L = 10
N_C = 16
N_OUT = 4

SC_RAD = 1
TC_RAD = N_RAD - SC_RAD
CHN = 128
NUM_WORKERS = 32
JGROUPS = CHN // 16
JSHIFT = JGROUPS.bit_length() - 1
FULL_CHUNKS = N_NODE // CHN
TAIL0 = FULL_CHUNKS * CHN
TAIL = N_NODE - TAIL0
TC_BN = 512


def _poly(a):
    s1 = a[1] * a[1]
    s2 = a[2] * a[2]
    s3 = a[3] * a[3]
    s4 = a[4] * a[4]
    s5 = a[5] * a[5]
    s6 = a[6] * a[6]
    s7 = a[7] * a[7]
    s8 = a[8] * a[8]
    s9 = a[9] * a[9]
    out1 = s1 + s2 + s3
    out2 = (s4 + s7 + s9) + 2.0 * (s5 + s6 + s8)
    cubes = a[4] * s4 + a[7] * s7 + a[9] * s9
    mixed = a[4] * (s5 + s6) + a[7] * (s5 + s8) + a[9] * (s6 + s8)
    triple = a[5] * a[6] * a[8]
    out3 = cubes + 3.0 * mixed + 6.0 * triple
    return a[0], out1, out2, out3


def _tc_body(x_ref, o_ref):
    a = [x_ref[:, l] for l in range(N_L)]
    o0, o1, o2, o3 = _poly(a)
    o_ref[:, 0] = o0
    o_ref[:, 1] = o1
    o_ref[:, 2] = o2
    o_ref[:, 3] = o3


if TC_RAD:
    _tc_call = pl.pallas_call(
        _tc_body,
        grid=((N_NODE + TC_BN - 1) // TC_BN,),
        in_specs=[
            pl.BlockSpec((TC_RAD, N_L, N_C, TC_BN), lambda i: (0, 0, 0, i)),
        ],
        out_specs=pl.BlockSpec((TC_RAD, N_OUT, N_C, TC_BN), lambda i: (0, 0, 0, i)),
        out_shape=jax.ShapeDtypeStruct(
            (N_RAD if SC_RAD else TC_RAD, N_OUT, N_C, N_NODE), jnp.float32
        ),
    )


def _sc_group_loop(xv, ov, ngroups, jgroups, jshift):
    @plsc.parallel_loop(0, ngroups, unroll=4)
    def group_body(m):
        if jgroups == 1:
            c = m
            j = 0
        else:
            c = m >> jshift
            j = (m & (jgroups - 1)) * 16
        a = [xv[l, c, pl.ds(j, 16)] for l in range(N_L)]
        o0, o1, o2, o3 = _poly(a)
        ov[0, c, pl.ds(j, 16)] = o0
        ov[1, c, pl.ds(j, 16)] = o1
        ov[2, c, pl.ds(j, 16)] = o2
        ov[3, c, pl.ds(j, 16)] = o3


def _sc_body(x_hbm, o_hbm, xv, ov, xt_v, ot_v):
    wid = lax.axis_index("s") * 2 + lax.axis_index("c")
    units = SC_RAD * FULL_CHUNKS
    my_units = (units - wid + NUM_WORKERS - 1) // NUM_WORKERS

    def unit_body(u, carry):
        unit = wid + u * NUM_WORKERS
        rr = unit // FULL_CHUNKS
        n0 = (unit % FULL_CHUNKS) * CHN
        pltpu.sync_copy(x_hbm.at[TC_RAD + rr, :, :, pl.ds(n0, CHN)], xv)
        _sc_group_loop(xv, ov, N_C * JGROUPS, JGROUPS, JSHIFT)
        pltpu.sync_copy(ov, o_hbm.at[rr, :, :, pl.ds(n0, CHN)])
        return carry

    lax.fori_loop(0, my_units, unit_body, 0)

    @pl.when(wid < SC_RAD)
    def _tail():
        pltpu.sync_copy(x_hbm.at[TC_RAD + wid, :, :, pl.ds(TAIL0, TAIL)], xt_v)
        _sc_group_loop(xt_v, ot_v, N_C, 1, 0)
        pltpu.sync_copy(ot_v, o_hbm.at[wid, :, :, pl.ds(TAIL0, TAIL)])


if SC_RAD:
    _sc_call = functools.partial(
        pl.kernel,
        out_type=jax.ShapeDtypeStruct((SC_RAD, N_OUT, N_C, N_NODE), jnp.float32),
        mesh=plsc.VectorSubcoreMesh(core_axis_name="c", subcore_axis_name="s"),
        scratch_types=[
            pltpu.VMEM((N_L, N_C, CHN), jnp.float32),
            pltpu.VMEM((N_OUT, N_C, CHN), jnp.float32),
            pltpu.VMEM((N_L, N_C, TAIL), jnp.float32),
            pltpu.VMEM((N_OUT, N_C, TAIL), jnp.float32),
        ],
    )(_sc_body)


@jax.jit
def kernel(node_attr):
    xt = jnp.transpose(node_attr, (1, 2, 3, 0))
    if TC_RAD and SC_RAD:
        tc_out = _tc_call(xt)
        sc_out = _sc_call(xt)
        yt = lax.dynamic_update_slice(tc_out, sc_out, (TC_RAD, 0, 0, 0))
    elif TC_RAD:
        yt = _tc_call(xt)
    else:
        yt = _sc_call(xt)
    return jnp.transpose(yt, (3, 0, 1, 2))


# speedup vs baseline: 1.9050x; 1.1132x over previous
import functools

import jax
import jax.numpy as jnp
from jax import lax
from jax.experimental import pallas as pl
from jax.experimental.pallas import tpu as pltpu
from jax.experimental.pallas import tpu_sc as plsc

N_NODE = 10000
N_RAD = 6
N_L = 10
N_C = 16
N_OUT = 4

SC_RAD = 1
TC_RAD = N_RAD - SC_RAD
CHN = 128
NUM_WORKERS = 32
JGROUPS = CHN // 16
JSHIFT = JGROUPS.bit_length() - 1
FULL_CHUNKS = N_NODE // CHN
TAIL0 = FULL_CHUNKS * CHN
TAIL = N_NODE - TAIL0
TC_BN = 1024


def _poly(a):
    s1 = a[1] * a[1]
    s2 = a[2] * a[2]
    s3 = a[3] * a[3]
    s4 = a[4] * a[4]
    s5 = a[5] * a[5]
    s6 = a[6] * a[6]
    s7 = a[7] * a[7]
    s8 = a[8] * a[8]
    s9 = a[9] * a[9]
    out1 = s1 + s2 + s3
    out2 = (s4 + s7 + s9) + 2.0 * (s5 + s6 + s8)
    cubes = a[4] * s4 + a[7] * s7 + a[9] * s9
    mixed = a[4] * (s5 + s6) + a[7] * (s5 + s8) + a[9] * (s6 + s8)
    triple = a[5] * a[6] * a[8]
    out3 = cubes + 3.0 * mixed + 6.0 * triple
    return a[0], out1, out2, out3


def _tc_body(x_ref, o_ref):
    a = [x_ref[:, l] for l in range(N_L)]
    o0, o1, o2, o3 = _poly(a)
    o_ref[:, 0] = o0
    o_ref[:, 1] = o1
    o_ref[:, 2] = o2
    o_ref[:, 3] = o3


if TC_RAD:
    _tc_call = pl.pallas_call(
        _tc_body,
        grid=((N_NODE + TC_BN - 1) // TC_BN,),
        in_specs=[
            pl.BlockSpec((TC_RAD, N_L, N_C, TC_BN), lambda i: (0, 0, 0, i)),
        ],
        out_specs=pl.BlockSpec((TC_RAD, N_OUT, N_C, TC_BN), lambda i: (0, 0, 0, i)),
        out_shape=jax.ShapeDtypeStruct(
            (N_RAD if SC_RAD else TC_RAD, N_OUT, N_C, N_NODE), jnp.float32
        ),
    )


def _sc_group_loop(xv, ov, ngroups, jgroups, jshift):
    @plsc.parallel_loop(0, ngroups, unroll=4)
    def group_body(m):
        if jgroups == 1:
            c = m
            j = 0
        else:
            c = m >> jshift
            j = (m & (jgroups - 1)) * 16
        a = [xv[l, c, pl.ds(j, 16)] for l in range(N_L)]
        o0, o1, o2, o3 = _poly(a)
        ov[0, c, pl.ds(j, 16)] = o0
        ov[1, c, pl.ds(j, 16)] = o1
        ov[2, c, pl.ds(j, 16)] = o2
        ov[3, c, pl.ds(j, 16)] = o3


def _sc_body(x_hbm, o_hbm, xv, ov, xt_v, ot_v):
    wid = lax.axis_index("s") * 2 + lax.axis_index("c")
    units = SC_RAD * FULL_CHUNKS
    my_units = (units - wid + NUM_WORKERS - 1) // NUM_WORKERS

    def unit_body(u, carry):
        unit = wid + u * NUM_WORKERS
        rr = unit // FULL_CHUNKS
        n0 = (unit % FULL_CHUNKS) * CHN
        pltpu.sync_copy(x_hbm.at[TC_RAD + rr, :, :, pl.ds(n0, CHN)], xv)
        _sc_group_loop(xv, ov, N_C * JGROUPS, JGROUPS, JSHIFT)
        pltpu.sync_copy(ov, o_hbm.at[rr, :, :, pl.ds(n0, CHN)])
        return carry

    lax.fori_loop(0, my_units, unit_body, 0)

    @pl.when(wid < SC_RAD)
    def _tail():
        pltpu.sync_copy(x_hbm.at[TC_RAD + wid, :, :, pl.ds(TAIL0, TAIL)], xt_v)
        _sc_group_loop(xt_v, ot_v, N_C, 1, 0)
        pltpu.sync_copy(ot_v, o_hbm.at[wid, :, :, pl.ds(TAIL0, TAIL)])


if SC_RAD:
    _sc_call = functools.partial(
        pl.kernel,
        out_type=jax.ShapeDtypeStruct((SC_RAD, N_OUT, N_C, N_NODE), jnp.float32),
        mesh=plsc.VectorSubcoreMesh(core_axis_name="c", subcore_axis_name="s"),
        scratch_types=[
            pltpu.VMEM((N_L, N_C, CHN), jnp.float32),
            pltpu.VMEM((N_OUT, N_C, CHN), jnp.float32),
            pltpu.VMEM((N_L, N_C, TAIL), jnp.float32),
            pltpu.VMEM((N_OUT, N_C, TAIL), jnp.float32),
        ],
    )(_sc_body)


@jax.jit
def kernel(node_attr):
    xt = jnp.transpose(node_attr, (1, 2, 3, 0))
    if TC_RAD and SC_RAD:
        tc_out = _tc_call(xt)
        sc_out = _sc_call(xt)
        yt = lax.dynamic_update_slice(tc_out, sc_out, (TC_RAD, 0, 0, 0))
    elif TC_RAD:
        yt = _tc_call(xt)
    else:
        yt = _sc_call(xt)
    return jnp.transpose(yt, (3, 0, 1, 2))
